# Initial kernel scaffold; baseline (speedup 1.0000x reference)
import jax
import jax.numpy as jnp
from jax.experimental import pallas as pl


def kernel(token_sequence, embedding_sequence, compression_rate):
    raise NotImplementedError("write your pallas kernel here")


# speedup vs baseline: 37.9735x; 37.9735x over previous
import jax
import jax.numpy as jnp
from jax.experimental import pallas as pl
from jax.experimental.pallas import tpu as pltpu


def _rank_mask_kernel(k_ref, tok_ref, out_ref):
    S = tok_ref.shape[1]
    k = k_ref[0]
    tok = tok_ref[...]
    pos = jax.lax.broadcasted_iota(jnp.int32, tok.shape, 1)
    neg = (tok > 0) & (pos >= 1) & (pos <= S - 2)
    negi = neg.astype(jnp.int32)
    cn = negi
    shift = 1
    while shift < S:
        rolled = pltpu.roll(cn, shift, 1)
        cn = cn + jnp.where(pos >= shift, rolled, 0)
        shift *= 2
    n_neg = cn[:, S - 1:S]
    rank = jnp.where(neg, cn, n_neg + pos - cn)
    rank = jnp.where(pos == 0, 0, rank)
    out_ref[...] = rank < k


def kernel(token_sequence, embedding_sequence, compression_rate):
    del embedding_sequence
    B, S = token_sequence.shape
    k = jnp.maximum(jnp.asarray(S * (1 - compression_rate)), 1).astype(jnp.int32)
    k_arr = jnp.reshape(k, (1,))
    y_hard = pl.pallas_call(
        _rank_mask_kernel,
        out_shape=jax.ShapeDtypeStruct((B, S), jnp.bool_),
        in_specs=[
            pl.BlockSpec(memory_space=pltpu.SMEM),
            pl.BlockSpec(memory_space=pltpu.VMEM),
        ],
        out_specs=pl.BlockSpec(memory_space=pltpu.VMEM),
    )(k_arr, token_sequence)
    return (y_hard, y_hard)
